# Initial kernel scaffold; baseline (speedup 1.0000x reference)
import jax
import jax.numpy as jnp
from jax.experimental import pallas as pl


def kernel(support, support_labels, query, W, b):
    raise NotImplementedError("write your pallas kernel here")


# speedup vs baseline: 1.4561x; 1.4561x over previous
import jax
import jax.numpy as jnp
from jax import lax
from jax.experimental import pallas as pl

_SEQ = 128
_D = 64
_C = 64
_K = _SEQ * _D
_BS = 256
_BQ = 256


def _support_body(labels_ref, x_ref, w_ref, sums_ref, counts_ref):
    i = pl.program_id(0)
    pooled = jnp.sum(x_ref[...], axis=2) * (1.0 / _SEQ)
    emb = jnp.dot(pooled, w_ref[...],
                  preferred_element_type=jnp.float32)
    lbl = labels_ref[0, 0, :]
    onehot = (lbl[:, None] == lax.broadcasted_iota(jnp.int32, (_BS, _C), 1)
              ).astype(jnp.float32)
    part_sums = lax.dot_general(onehot, emb, (((0,), (0,)), ((), ())),
                                preferred_element_type=jnp.float32)
    ones_col = jnp.ones((_BS, 1), jnp.float32)
    part_counts = lax.dot_general(onehot, ones_col, (((0,), (0,)), ((), ())),
                                  preferred_element_type=jnp.float32)

    @pl.when(i == 0)
    def _():
        sums_ref[...] = part_sums
        counts_ref[...] = part_counts

    @pl.when(i > 0)
    def _():
        sums_ref[...] += part_sums
        counts_ref[...] += part_counts


def _query_body(x_ref, w_ref, b_ref, sums_ref, counts_ref,
                logits_t_ref, protos_ref):
    j = pl.program_id(0)
    counts = counts_ref[...]
    denom = jnp.maximum(counts, 1.0)
    protos = (sums_ref[...] + counts * b_ref[...]) / denom

    @pl.when(j == 0)
    def _():
        protos_ref[...] = protos

    pooled = jnp.sum(x_ref[...], axis=2) * (1.0 / _SEQ)
    qe = jnp.dot(pooled, w_ref[...],
                 preferred_element_type=jnp.float32) + b_ref[...]
    p2 = jnp.sum(protos * protos, axis=1, keepdims=True)
    ones_row = jnp.ones((1, _D), jnp.float32)
    q2t = lax.dot_general(ones_row, qe * qe, (((1,), (1,)), ((), ())),
                          preferred_element_type=jnp.float32)
    cross_t = lax.dot_general(protos, qe, (((1,), (1,)), ((), ())),
                              preferred_element_type=jnp.float32)
    logits_t_ref[...] = -(p2 + q2t - 2.0 * cross_t + 1e-8)


@jax.jit
def kernel(support, support_labels, query, W, b):
    n_sup = support.shape[0]
    n_q = query.shape[0]
    nbs = n_sup // _BS
    nbq = n_q // _BQ
    sup_t = support.transpose(0, 2, 1)
    q_t = query.transpose(0, 2, 1)
    labels = support_labels.astype(jnp.int32).reshape(nbs, 1, _BS)
    b_row = b.reshape(1, _D)

    sums, counts = pl.pallas_call(
        _support_body,
        grid=(nbs,),
        in_specs=[
            pl.BlockSpec((1, 1, _BS), lambda i: (i, 0, 0)),
            pl.BlockSpec((_BS, _D, _SEQ), lambda i: (i, 0, 0)),
            pl.BlockSpec((_D, _D), lambda i: (0, 0)),
        ],
        out_specs=[
            pl.BlockSpec((_C, _D), lambda i: (0, 0)),
            pl.BlockSpec((_C, 1), lambda i: (0, 0)),
        ],
        out_shape=[
            jax.ShapeDtypeStruct((_C, _D), jnp.float32),
            jax.ShapeDtypeStruct((_C, 1), jnp.float32),
        ],
    )(labels, sup_t, W)

    logits_t, protos = pl.pallas_call(
        _query_body,
        grid=(nbq,),
        in_specs=[
            pl.BlockSpec((_BQ, _D, _SEQ), lambda j: (j, 0, 0)),
            pl.BlockSpec((_D, _D), lambda j: (0, 0)),
            pl.BlockSpec((1, _D), lambda j: (0, 0)),
            pl.BlockSpec((_C, _D), lambda j: (0, 0)),
            pl.BlockSpec((_C, 1), lambda j: (0, 0)),
        ],
        out_specs=[
            pl.BlockSpec((_C, _BQ), lambda j: (0, j)),
            pl.BlockSpec((_C, _D), lambda j: (0, 0)),
        ],
        out_shape=[
            jax.ShapeDtypeStruct((_C, n_q), jnp.float32),
            jax.ShapeDtypeStruct((_C, _D), jnp.float32),
        ],
    )(q_t, W, b_row, sums, counts)

    return (logits_t.T, protos)
